# baseline (speedup 1.0000x reference)
import jax
import jax.numpy as jnp
from jax import lax
from jax.experimental import pallas as pl
from jax.experimental.pallas import tpu as pltpu
from jax.experimental.pallas import tpu_sc as plsc

BATCH = 4096
FAST_DIM = 64
VOCAB = 100000

_NC = 2
_NS = 16
_NW = _NC * _NS
_D_PER_W = FAST_DIM // _NW
_LANES = 16

_BN = 1024
_NSTEPS = pl.cdiv(VOCAB, _BN)
_TAIL = VOCAB - (_NSTEPS - 1) * _BN
_NBUF = 2


def _sc_colgather_body(idx_hbm, tablet_hbm, out_hbm, idx_v, row_v, gath_v, sem):
    wid = lax.axis_index("s") * _NC + lax.axis_index("c")
    pltpu.sync_copy(idx_hbm, idx_v)
    for r in range(_D_PER_W):
        d = wid * _D_PER_W + r
        pltpu.async_copy(tablet_hbm.at[d], row_v, sem).wait()

        def body(j, _):
            iv = idx_v[pl.ds(j * _LANES, _LANES)]
            gath_v[pl.ds(j * _LANES, _LANES)] = plsc.load_gather(row_v, [iv])
            return 0

        lax.fori_loop(0, BATCH // _LANES, body, 0)
        pltpu.sync_copy(gath_v, out_hbm.at[d])


def _sc_gather_t(last_tok, tablet):
    mesh = plsc.VectorSubcoreMesh(core_axis_name="c", subcore_axis_name="s")
    return pl.kernel(
        _sc_colgather_body,
        mesh=mesh,
        out_type=jax.ShapeDtypeStruct((FAST_DIM, BATCH), jnp.float32),
        scratch_types=[
            pltpu.VMEM((BATCH,), jnp.int32),
            pltpu.VMEM((VOCAB,), jnp.float32),
            pltpu.VMEM((BATCH,), jnp.float32),
            pltpu.SemaphoreType.DMA,
        ],
        compiler_params=pltpu.CompilerParams(needs_layout_passes=False),
    )(last_tok, tablet)


def _slab_copy(acc, out_hbm, sems, slot, step):
    return pltpu.make_async_copy(
        acc.at[slot],
        out_hbm.at[pl.ds(step * _BN, _BN)],
        sems.at[slot],
    )


def _tail_copy(acc, out_hbm, sems, slot):
    return pltpu.make_async_copy(
        acc.at[slot, pl.ds(0, _TAIL)],
        out_hbm.at[pl.ds((_NSTEPS - 1) * _BN, _TAIL)],
        sems.at[slot],
    )


def _mm_body(embt_ref, w_ref, b_ref, out_hbm, acc, sems):
    i = pl.program_id(0)
    slot = lax.rem(i, _NBUF)

    @pl.when(i >= _NBUF)
    def _():
        _slab_copy(acc, out_hbm, sems, slot, i - _NBUF).wait()

    acc[slot] = (
        lax.dot_general(
            w_ref[...], embt_ref[...],
            (((0,), (0,)), ((), ())),
            preferred_element_type=jnp.float32,
        )
        + b_ref[...].T
    )

    @pl.when(i < _NSTEPS - 1)
    def _():
        _slab_copy(acc, out_hbm, sems, slot, i).start()

    @pl.when(i == _NSTEPS - 1)
    def _():
        _tail_copy(acc, out_hbm, sems, slot).start()
        for k in range(_NBUF):
            s = _NSTEPS - _NBUF + k
            if s == _NSTEPS - 1:
                _tail_copy(acc, out_hbm, sems, s % _NBUF).wait()
            else:
                _slab_copy(acc, out_hbm, sems, s % _NBUF, s).wait()


def _tc_project_t(embt, W, brow):
    return pl.pallas_call(
        _mm_body,
        grid=(_NSTEPS,),
        in_specs=[
            pl.BlockSpec((FAST_DIM, BATCH), lambda i: (0, 0)),
            pl.BlockSpec((FAST_DIM, _BN), lambda i: (0, i)),
            pl.BlockSpec((1, _BN), lambda i: (0, i)),
        ],
        out_specs=pl.BlockSpec(memory_space=pltpu.HBM),
        out_shape=jax.ShapeDtypeStruct((VOCAB, BATCH), jnp.float32),
        scratch_shapes=[
            pltpu.VMEM((_NBUF, _BN, BATCH), jnp.float32),
            pltpu.SemaphoreType.DMA((_NBUF,)),
        ],
        compiler_params=pltpu.CompilerParams(
            dimension_semantics=("arbitrary",),
        ),
    )(embt, W, brow)


def kernel(x, emb_table, W, b):
    last_tok = x[:, -1].astype(jnp.int32)
    embt = _sc_gather_t(last_tok, emb_table.T)
    logits_t = _tc_project_t(embt, W, b.reshape(1, VOCAB))
    return logits_t.T
